# baseline (speedup 1.0000x reference)
import jax
import jax.numpy as jnp
from jax.experimental import pallas as pl

_BB = 2048
_BD = 128


def _lookup_block(t_ref, tbl_ref, out_ref):
    tb = t_ref[0, 0, :].reshape(_BB, 1)
    v = tbl_ref.shape[0]
    col = jax.lax.broadcasted_iota(jnp.int32, (_BB, v), 1)
    oh = (tb == col).astype(jnp.float32)
    out_ref[:, :] = jnp.dot(oh, tbl_ref[:], preferred_element_type=jnp.float32)


def kernel(t, embed):
    B = t.shape[0]
    V, D = embed.shape
    nb = B // _BB
    nd = D // _BD
    t3 = t.astype(jnp.int32).reshape(nb, 1, _BB)
    return pl.pallas_call(
        _lookup_block,
        grid=(nb, nd),
        in_specs=[
            pl.BlockSpec((1, 1, _BB), lambda i, j: (i, 0, 0)),
            pl.BlockSpec((V, _BD), lambda i, j: (0, j)),
        ],
        out_specs=pl.BlockSpec((_BB, _BD), lambda i, j: (i, j)),
        out_shape=jax.ShapeDtypeStruct((B, D), jnp.float32),
    )(t3, embed)
